# Initial kernel scaffold; baseline (speedup 1.0000x reference)
import jax
import jax.numpy as jnp
from jax.experimental import pallas as pl


def kernel(x, table):
    raise NotImplementedError("write your pallas kernel here")


# speedup vs baseline: 1.3036x; 1.3036x over previous
import jax
import jax.numpy as jnp
from jax import lax
from jax.experimental import pallas as pl
from jax.experimental.pallas import tpu as pltpu
from jax.experimental.pallas import tpu_sc as plsc

_NUM_CORES = 2
_NUM_SUBCORES = 16
_CHUNK = 80


def kernel(x, table):
    batch, seq = x.shape
    vocab, emb = table.shape
    n = batch * seq
    nw = _NUM_CORES * _NUM_SUBCORES
    npw = n // nw
    nch = npw // _CHUNK

    idx = x.reshape(n).astype(jnp.int32)
    mesh = plsc.VectorSubcoreMesh(core_axis_name="c", subcore_axis_name="s")

    @pl.kernel(
        out_type=jax.ShapeDtypeStruct((n, emb), jnp.float32),
        mesh=mesh,
        scratch_types=[
            pltpu.VMEM((npw,), jnp.int32),
            pltpu.VMEM((2, _CHUNK, emb), jnp.float32),
            pltpu.SemaphoreType.DMA((2,)),
            pltpu.SemaphoreType.DMA((2,)),
        ],
    )
    def k(table_hbm, i_hbm, o_hbm, idx_v, buf, gsem, osem):
        wid = lax.axis_index("s") * _NUM_CORES + lax.axis_index("c")
        base = wid * npw
        pltpu.sync_copy(i_hbm.at[pl.ds(base, npw)], idx_v)

        def g_copy(g, b):
            return pltpu.make_async_copy(
                table_hbm.at[idx_v.at[pl.ds(g * _CHUNK, _CHUNK)]],
                buf.at[b],
                gsem.at[b],
            )

        def o_copy(g, b):
            return pltpu.make_async_copy(
                buf.at[b],
                o_hbm.at[pl.ds(base + g * _CHUNK, _CHUNK)],
                osem.at[b],
            )

        g_copy(0, 0).start()
        g_copy(0, 0).wait()
        g_copy(1, 1).start()
        o_copy(0, 0).start()

        @pl.loop(0, (nch - 2) // 2)
        def _(c):
            for b in (0, 1):
                g = 2 + 2 * c + b
                o_copy(g - 2, b).wait()
                g_copy(g, b).start()
                g_copy(g - 1, 1 - b).wait()
                o_copy(g - 1, 1 - b).start()

        g_copy(nch - 1, 1).wait()
        o_copy(nch - 1, 1).start()
        o_copy(nch - 2, 0).wait()
        o_copy(nch - 1, 1).wait()

    return k(table, idx).reshape(batch, seq, emb)


# speedup vs baseline: 1.3075x; 1.0030x over previous
import jax
import jax.numpy as jnp
from jax import lax
from jax.experimental import pallas as pl
from jax.experimental.pallas import tpu as pltpu
from jax.experimental.pallas import tpu_sc as plsc

_NUM_CORES = 2
_NUM_SUBCORES = 16
_CHUNK = 80


def kernel(x, table):
    batch, seq = x.shape
    vocab, emb = table.shape
    n = batch * seq
    nw = _NUM_CORES * _NUM_SUBCORES
    npw = n // nw
    nch = npw // _CHUNK

    idx = x.reshape(n).astype(jnp.int32)
    mesh = plsc.VectorSubcoreMesh(core_axis_name="c", subcore_axis_name="s")

    nbuf = 3

    @pl.kernel(
        out_type=jax.ShapeDtypeStruct((n, emb), jnp.float32),
        mesh=mesh,
        scratch_types=[
            pltpu.VMEM((npw,), jnp.int32),
            pltpu.VMEM((nbuf, _CHUNK, emb), jnp.float32),
            pltpu.SemaphoreType.DMA((nbuf,)),
            pltpu.SemaphoreType.DMA((nbuf,)),
        ],
    )
    def k(table_hbm, i_hbm, o_hbm, idx_v, buf, gsem, osem):
        wid = lax.axis_index("s") * _NUM_CORES + lax.axis_index("c")
        base = wid * npw
        pltpu.sync_copy(i_hbm.at[pl.ds(base, npw)], idx_v)

        def g_copy(g, b):
            return pltpu.make_async_copy(
                table_hbm.at[idx_v.at[pl.ds(g * _CHUNK, _CHUNK)]],
                buf.at[b],
                gsem.at[b],
            )

        def o_copy(g, b):
            return pltpu.make_async_copy(
                buf.at[b],
                o_hbm.at[pl.ds(base + g * _CHUNK, _CHUNK)],
                osem.at[b],
            )

        g_copy(0, 0).start()
        g_copy(1, 1).start()
        g_copy(0, 0).wait()
        o_copy(0, 0).start()
        g_copy(2, 2).start()

        @pl.loop(1, nch - 2)
        def _(g):
            b = g % nbuf
            g_copy(g, b).wait()
            o_copy(g, b).start()
            o_copy(g - 1, (g - 1) % nbuf).wait()
            g_copy(g + 2, (g + 2) % nbuf).start()

        for g in (nch - 2, nch - 1):
            b = g % nbuf
            g_copy(g, b).wait()
            o_copy(g, b).start()
            o_copy(g - 1, (g - 1) % nbuf).wait()
        o_copy(nch - 1, (nch - 1) % nbuf).wait()

    return k(table, idx).reshape(batch, seq, emb)
